# baseline (speedup 1.0000x reference)
import functools

import jax
import jax.numpy as jnp
from jax.experimental import pallas as pl
from jax.experimental.pallas import tpu as pltpu

_P = 16
_NCHUNK = 4
_AP = 32


def _mlp_kernel(xp_ref, w1_ref, b1_ref, w2_ref, b2_ref, o_ref):
    k = w1_ref.shape[0]
    a = o_ref.shape[-1] // _P
    r = xp_ref.shape[0]
    rc = r // _NCHUNK
    w1a = jnp.concatenate([w1_ref[...], b1_ref[...]], axis=0)
    w1a = w1a.astype(jnp.bfloat16)
    w2 = w2_ref[...][:, :_AP]
    b2pk = jnp.concatenate([b2_ref[0:1, :a]] * _P, axis=-1)
    for c in range(_NCHUNK):
        xp = xp_ref[c * rc:(c + 1) * rc, :]
        x = jnp.concatenate(
            [xp[:, s * k:(s + 1) * k] for s in range(_P)], axis=0
        )
        ones = jnp.ones((x.shape[0], 1), x.dtype)
        xa = jnp.concatenate([x, ones], axis=-1).astype(jnp.bfloat16)
        ht = jax.lax.dot_general(
            w1a, xa, (((0,), (1,)), ((), ())),
            preferred_element_type=jnp.float32,
        )
        ht = jnp.maximum(ht, 0.0)
        qt = jax.lax.dot_general(
            w2, ht, (((0,), (0,)), ((), ())),
            preferred_element_type=jnp.float32,
        )
        o_ref[c * rc:(c + 1) * rc, :] = jnp.concatenate(
            [qt[:a, s * rc:(s + 1) * rc].T for s in range(_P)], axis=-1
        ) + b2pk


@functools.partial(jax.jit, static_argnames=("num_actions", "tb"))
def _forward(x, w1p, b1p, w2p, b2p, *, num_actions, tb):
    B, K = x.shape
    H = w1p.shape[1]
    Ap = w2p.shape[1]
    A = num_actions
    R = tb // _P

    xp = x.reshape(B // _P, _P * K)
    grid = (B // tb,)
    cost = pl.CostEstimate(
        flops=2 * B * (K * H + H * _AP),
        transcendentals=0,
        bytes_accessed=4 * (B * K + B * A + K * H + H * _AP + H + _AP),
    )

    out = pl.pallas_call(
        _mlp_kernel,
        out_shape=jax.ShapeDtypeStruct((B // _P, _P * A), x.dtype),
        grid=grid,
        in_specs=[
            pl.BlockSpec((R, _P * K), lambda i: (i, 0)),
            pl.BlockSpec((K, H), lambda i: (0, 0)),
            pl.BlockSpec((1, H), lambda i: (0, 0)),
            pl.BlockSpec((H, Ap), lambda i: (0, 0)),
            pl.BlockSpec((1, Ap), lambda i: (0, 0)),
        ],
        out_specs=pl.BlockSpec((R, _P * A), lambda i: (i, 0)),
        compiler_params=pltpu.CompilerParams(
            dimension_semantics=("parallel",),
            vmem_limit_bytes=96 * 1024 * 1024,
        ),
        cost_estimate=cost,
    )(xp, w1p, b1p, w2p, b2p)
    return out.reshape(B, A)


def kernel(x, w1p, b1p, w2p, b2p):
    return _forward(x, w1p, b1p, w2p, b2p, num_actions=18, tb=8192)


# speedup vs baseline: 1.2100x; 1.2100x over previous
import functools

import jax
import jax.numpy as jnp
from jax.experimental import pallas as pl
from jax.experimental.pallas import tpu as pltpu

_P = 16
_NCHUNK = 4
_AP = 32


def _mlp_kernel(x_ref, w1_ref, b1_ref, w2_ref, b2_ref, o_ref):
    k = w1_ref.shape[0]
    a = o_ref.shape[-1] // _P
    tb = x_ref.shape[0]
    tbc = tb // _NCHUNK
    ro = tbc // _P
    w1a = jnp.concatenate([w1_ref[...], b1_ref[...]], axis=0)
    w1a = w1a.astype(jnp.bfloat16)
    w2 = w2_ref[...][:, :_AP]
    b2pk = jnp.concatenate([b2_ref[0:1, :a]] * _P, axis=-1)
    for c in range(_NCHUNK):
        x_c = x_ref[c * tbc:(c + 1) * tbc, :]
        x_cat = x_c.reshape(ro, _P, k).transpose(1, 0, 2).reshape(tbc, k)
        ones = jnp.ones((x_cat.shape[0], 1), x_cat.dtype)
        xa = jnp.concatenate([x_cat, ones], axis=-1).astype(jnp.bfloat16)
        ht = jax.lax.dot_general(
            w1a, xa, (((0,), (1,)), ((), ())),
            preferred_element_type=jnp.float32,
        )
        ht = jnp.maximum(ht, 0.0)
        qt = jax.lax.dot_general(
            w2, ht, (((0,), (0,)), ((), ())),
            preferred_element_type=jnp.float32,
        )
        o_ref[c * ro:(c + 1) * ro, :] = jnp.concatenate(
            [qt[:a, s * ro:(s + 1) * ro].T for s in range(_P)], axis=-1
        ) + b2pk


@functools.partial(jax.jit, static_argnames=("num_actions", "tb"))
def _forward(x, w1p, b1p, w2p, b2p, *, num_actions, tb):
    B, K = x.shape
    H = w1p.shape[1]
    Ap = w2p.shape[1]
    A = num_actions

    grid = (B // tb,)
    cost = pl.CostEstimate(
        flops=2 * B * (K * H + H * _AP),
        transcendentals=0,
        bytes_accessed=4 * (B * K + B * A + K * H + H * _AP + H + _AP),
    )

    out = pl.pallas_call(
        _mlp_kernel,
        out_shape=jax.ShapeDtypeStruct((B // _P, _P * A), x.dtype),
        grid=grid,
        in_specs=[
            pl.BlockSpec((tb, K), lambda i: (i, 0)),
            pl.BlockSpec((K, H), lambda i: (0, 0)),
            pl.BlockSpec((1, H), lambda i: (0, 0)),
            pl.BlockSpec((H, Ap), lambda i: (0, 0)),
            pl.BlockSpec((1, Ap), lambda i: (0, 0)),
        ],
        out_specs=pl.BlockSpec((tb // _P, _P * A), lambda i: (i, 0)),
        compiler_params=pltpu.CompilerParams(
            dimension_semantics=("arbitrary",),
            vmem_limit_bytes=96 * 1024 * 1024,
        ),
        cost_estimate=cost,
    )(x, w1p, b1p, w2p, b2p)
    return out.reshape(B, A)


def kernel(x, w1p, b1p, w2p, b2p):
    return _forward(x, w1p, b1p, w2p, b2p, num_actions=18, tb=8192)


# speedup vs baseline: 1.7317x; 1.4311x over previous
import functools

import jax
import jax.numpy as jnp
from jax.experimental import pallas as pl
from jax.experimental.pallas import tpu as pltpu

_NSUB = 4
_AP = 32


def _mlp_kernel(nsteps, a, x_hbm, w1_ref, b1_ref, w2_ref, b2_ref, o_hbm,
                xbuf, obuf, xsem, osem):
    i = pl.program_id(0)
    tb = xbuf.shape[1]
    rs = tb // _NSUB
    slot = jax.lax.rem(i, 2)
    nslot = jax.lax.rem(i + 1, 2)

    @pl.when(i == 0)
    def _():
        for j in range(_NSUB):
            pltpu.make_async_copy(
                x_hbm.at[pl.ds(j * rs, rs), :],
                xbuf.at[0, pl.ds(j * rs, rs), :], xsem.at[0, j]).start()

    @pl.when(i + 1 < nsteps)
    def _():
        base = (i + 1) * tb
        for j in range(_NSUB):
            pltpu.make_async_copy(
                x_hbm.at[pl.ds(base + j * rs, rs), :],
                xbuf.at[nslot, pl.ds(j * rs, rs), :],
                xsem.at[nslot, j]).start()

    w1a = jnp.concatenate([w1_ref[...], b1_ref[...]], axis=0)
    w1a = w1a.astype(jnp.bfloat16)
    w2 = w2_ref[...][:, :_AP]
    b2 = b2_ref[0:1, :a]

    for c in range(_NSUB):
        pltpu.make_async_copy(
            xbuf.at[slot, pl.ds(c * rs, rs), :],
            xbuf.at[slot, pl.ds(c * rs, rs), :], xsem.at[slot, c]).wait()
        xc = xbuf[slot, c * rs:(c + 1) * rs, :]
        ones = jnp.ones((rs, 1), xc.dtype)
        xa = jnp.concatenate([xc, ones], axis=-1).astype(jnp.bfloat16)
        ht = jax.lax.dot_general(
            w1a, xa, (((0,), (1,)), ((), ())),
            preferred_element_type=jnp.float32,
        )
        ht = jnp.maximum(ht, 0.0)
        qt = jax.lax.dot_general(
            w2, ht, (((0,), (0,)), ((), ())),
            preferred_element_type=jnp.float32,
        )
        q = qt[:a, :].T + b2

        @pl.when(i >= 2)
        def _():
            pltpu.make_async_copy(
                obuf.at[slot, pl.ds(c * rs, rs), :],
                obuf.at[slot, pl.ds(c * rs, rs), :], osem.at[slot, c]).wait()

        obuf[slot, c * rs:(c + 1) * rs, :] = q
        pltpu.make_async_copy(
            obuf.at[slot, pl.ds(c * rs, rs), :],
            o_hbm.at[pl.ds(i * tb + c * rs, rs), :], osem.at[slot, c]).start()

    @pl.when(i == nsteps - 1)
    def _():
        for j in range(_NSUB):
            pltpu.make_async_copy(
                obuf.at[slot, pl.ds(j * rs, rs), :],
                obuf.at[slot, pl.ds(j * rs, rs), :], osem.at[slot, j]).wait()

    @pl.when((i == nsteps - 1) & (nsteps > 1))
    def _():
        for j in range(_NSUB):
            pltpu.make_async_copy(
                obuf.at[nslot, pl.ds(j * rs, rs), :],
                obuf.at[nslot, pl.ds(j * rs, rs), :], osem.at[nslot, j]).wait()


@functools.partial(jax.jit, static_argnames=("num_actions", "tb"))
def _forward(x, w1p, b1p, w2p, b2p, *, num_actions, tb):
    B, K = x.shape
    H = w1p.shape[1]
    Ap = w2p.shape[1]
    A = num_actions
    nsteps = B // tb

    cost = pl.CostEstimate(
        flops=2 * B * (K * H + H * _AP),
        transcendentals=0,
        bytes_accessed=4 * (B * K + B * A + K * H + H * _AP + H + _AP),
    )

    return pl.pallas_call(
        functools.partial(_mlp_kernel, nsteps, A),
        out_shape=jax.ShapeDtypeStruct((B, A), x.dtype),
        grid=(nsteps,),
        in_specs=[
            pl.BlockSpec(memory_space=pl.ANY),
            pl.BlockSpec((K, H), lambda i: (0, 0)),
            pl.BlockSpec((1, H), lambda i: (0, 0)),
            pl.BlockSpec((H, Ap), lambda i: (0, 0)),
            pl.BlockSpec((1, Ap), lambda i: (0, 0)),
        ],
        out_specs=pl.BlockSpec(memory_space=pl.ANY),
        scratch_shapes=[
            pltpu.VMEM((2, tb, K), jnp.float32),
            pltpu.VMEM((2, tb, A), jnp.float32),
            pltpu.SemaphoreType.DMA((2, _NSUB)),
            pltpu.SemaphoreType.DMA((2, _NSUB)),
        ],
        compiler_params=pltpu.CompilerParams(
            dimension_semantics=("arbitrary",),
            vmem_limit_bytes=96 * 1024 * 1024,
        ),
        cost_estimate=cost,
    )(x, w1p, b1p, w2p, b2p)


def kernel(x, w1p, b1p, w2p, b2p):
    return _forward(x, w1p, b1p, w2p, b2p, num_actions=18, tb=8192)


# speedup vs baseline: 1.9079x; 1.1017x over previous
import functools

import jax
import jax.numpy as jnp
from jax.experimental import pallas as pl
from jax.experimental.pallas import tpu as pltpu

_NCHUNK = 4
_AP = 32


def _mlp_kernel(x_ref, w1_ref, b1_ref, w2_ref, b2_ref, o_ref):
    tb = x_ref.shape[0]
    a = o_ref.shape[-1]
    rs = tb // _NCHUNK
    w1a = jnp.concatenate([w1_ref[...], b1_ref[...]], axis=0)
    w1a = w1a.astype(jnp.bfloat16)
    w2 = w2_ref[...][:, :_AP]
    b2 = b2_ref[0:1, :a]
    for c in range(_NCHUNK):
        xc = x_ref[c * rs:(c + 1) * rs, :]
        ones = jnp.ones((rs, 1), xc.dtype)
        xa = jnp.concatenate([xc, ones], axis=-1).astype(jnp.bfloat16)
        ht = jax.lax.dot_general(
            w1a, xa, (((0,), (1,)), ((), ())),
            preferred_element_type=jnp.float32,
        )
        ht = jnp.maximum(ht, 0.0)
        qt = jax.lax.dot_general(
            w2, ht, (((0,), (0,)), ((), ())),
            preferred_element_type=jnp.float32,
        )
        o_ref[c * rs:(c + 1) * rs, :] = qt[:a, :].T + b2


@functools.partial(jax.jit, static_argnames=("num_actions", "tb"))
def _forward(x, w1p, b1p, w2p, b2p, *, num_actions, tb):
    B, K = x.shape
    H = w1p.shape[1]
    Ap = w2p.shape[1]
    A = num_actions

    grid = (B // tb,)
    cost = pl.CostEstimate(
        flops=2 * B * (K * H + H * _AP),
        transcendentals=0,
        bytes_accessed=4 * (B * K + B * A + K * H + H * _AP + H + _AP),
    )

    return pl.pallas_call(
        _mlp_kernel,
        out_shape=jax.ShapeDtypeStruct((B, A), x.dtype),
        grid=grid,
        in_specs=[
            pl.BlockSpec((tb, K), lambda i: (i, 0)),
            pl.BlockSpec((K, H), lambda i: (0, 0)),
            pl.BlockSpec((1, H), lambda i: (0, 0)),
            pl.BlockSpec((H, Ap), lambda i: (0, 0)),
            pl.BlockSpec((1, Ap), lambda i: (0, 0)),
        ],
        out_specs=pl.BlockSpec((tb, A), lambda i: (i, 0)),
        compiler_params=pltpu.CompilerParams(
            dimension_semantics=("arbitrary",),
            vmem_limit_bytes=96 * 1024 * 1024,
        ),
        cost_estimate=cost,
    )(x, w1p, b1p, w2p, b2p)


def kernel(x, w1p, b1p, w2p, b2p):
    return _forward(x, w1p, b1p, w2p, b2p, num_actions=18, tb=8192)


# speedup vs baseline: 1.9226x; 1.0077x over previous
import functools

import jax
import jax.numpy as jnp
from jax.experimental import pallas as pl
from jax.experimental.pallas import tpu as pltpu

_NCHUNK = 4
_AP = 32


def _mlp_kernel(x_ref, w1_ref, b1_ref, w2_ref, b2_ref, o_ref):
    tb = x_ref.shape[0]
    a = o_ref.shape[-1]
    rs = tb // _NCHUNK
    w1a = jnp.concatenate([w1_ref[...], b1_ref[...]], axis=0)
    w1a = w1a.astype(jnp.bfloat16)
    w2 = w2_ref[...][:, :_AP]
    b2 = b2_ref[0:1, :a]
    for c in range(_NCHUNK):
        xc = x_ref[c * rs:(c + 1) * rs, :]
        ones = jnp.ones((rs, 1), xc.dtype)
        xa = jnp.concatenate([xc, ones], axis=-1).astype(jnp.bfloat16)
        ht = jax.lax.dot_general(
            w1a, xa, (((0,), (1,)), ((), ())),
            preferred_element_type=jnp.float32,
        )
        ht = jnp.maximum(ht, 0.0)
        qt = jax.lax.dot_general(
            w2, ht, (((0,), (0,)), ((), ())),
            preferred_element_type=jnp.float32,
        )
        o_ref[c * rs:(c + 1) * rs, :] = qt[:a, :].T + b2


@functools.partial(jax.jit, static_argnames=("num_actions", "tb"))
def _forward(x, w1p, b1p, w2p, b2p, *, num_actions, tb):
    B, K = x.shape
    H = w1p.shape[1]
    Ap = w2p.shape[1]
    A = num_actions

    grid = (B // tb,)
    cost = pl.CostEstimate(
        flops=2 * B * (K * H + H * _AP),
        transcendentals=0,
        bytes_accessed=4 * (B * K + B * A + K * H + H * _AP + H + _AP),
    )

    return pl.pallas_call(
        _mlp_kernel,
        out_shape=jax.ShapeDtypeStruct((B, A), x.dtype),
        grid=grid,
        in_specs=[
            pl.BlockSpec((tb, K), lambda i: (i, 0)),
            pl.BlockSpec((K, H), lambda i: (0, 0)),
            pl.BlockSpec((1, H), lambda i: (0, 0)),
            pl.BlockSpec((H, Ap), lambda i: (0, 0)),
            pl.BlockSpec((1, Ap), lambda i: (0, 0)),
        ],
        out_specs=pl.BlockSpec((tb, A), lambda i: (i, 0)),
        compiler_params=pltpu.CompilerParams(
            dimension_semantics=("arbitrary",),
            vmem_limit_bytes=96 * 1024 * 1024,
        ),
        cost_estimate=cost,
    )(x, w1p, b1p, w2p, b2p)


def kernel(x, w1p, b1p, w2p, b2p):
    return _forward(x, w1p, b1p, w2p, b2p, num_actions=18, tb=16384)
